# baseline (speedup 1.0000x reference)
import jax
import jax.numpy as jnp
from jax.experimental import pallas as pl
from jax.experimental.pallas import tpu as pltpu

DIM = 1024
N_EXP = 16
DIM_GCN = 256
THRESH = 0.8
NTOK = 256
NQ = 4
QROWS = DIM // NQ
_HI = jax.lax.Precision.HIGHEST


def _moegnn_body(x_ref, XT_ref, Wm_ref, W0_ref, W1_ref, W2_ref, Wp_ref,
                 out_ref, u0_acc, e_s, ew0_acc):
    f32 = jnp.float32
    j = pl.program_id(0)

    Wm_q = Wm_ref[...]
    W0_q = W0_ref[pl.ds(j * QROWS, QROWS), :]
    xf_q = jnp.maximum(
        jax.lax.dot_general(x_ref[...], Wm_q, (((1,), (1,)), ((), ())),
                            preferred_element_type=f32), 0.0)
    p_u0 = jnp.dot(xf_q, W0_q, preferred_element_type=f32)
    e_q = jnp.maximum(
        jax.lax.dot_general(XT_ref[...], Wm_q, (((1,), (1,)), ((), ())),
                            preferred_element_type=f32), 0.0)
    e_s[:, pl.ds(j * QROWS, QROWS)] = e_q
    p_ew0 = jnp.dot(e_q, W0_q, preferred_element_type=f32)

    @pl.when(j == 0)
    def _init():
        u0_acc[...] = p_u0
        ew0_acc[...] = p_ew0

    @pl.when(j > 0)
    def _accum():
        u0_acc[...] += p_u0
        ew0_acc[...] += p_ew0

    @pl.when(j == NQ - 1)
    def _tail():
        W1 = W1_ref[...]
        ee = e_s[...]

        v_row = jax.lax.dot_general(Wp_ref[...], W2_ref[...],
                                    (((0,), (1,)), ((), ())),
                                    preferred_element_type=f32)

        G = jax.lax.dot_general(ee, ee, (((1,), (1,)), ((), ())),
                                preferred_element_type=f32)
        nrm2 = jnp.sum(ee * ee, axis=1, keepdims=True)
        nrm_col = jnp.maximum(jnp.sqrt(nrm2), 1e-8)
        nrm_row = jnp.transpose(nrm_col)

        ri = jax.lax.broadcasted_iota(jnp.int32, (N_EXP, N_EXP), 0)
        ci = jax.lax.broadcasted_iota(jnp.int32, (N_EXP, N_EXP), 1)
        eye = jnp.where(ri == ci, 1.0, 0.0)
        cos = G / (nrm_col * nrm_row)
        ind = (cos > THRESH).astype(f32)
        lower = jnp.where(ri > ci, ind, 0.0)
        upper = jnp.where(ri < ci, ind, 0.0)
        dinv_col = jax.lax.rsqrt(2.0 + jnp.sum(lower, axis=1, keepdims=True))
        dinv_row = jax.lax.rsqrt(2.0 + jnp.sum(upper, axis=0, keepdims=True))
        A = dinv_col * dinv_row * (lower + eye)

        C0 = jnp.dot(A, ew0_acc[...], preferred_element_type=f32)
        C1 = jnp.dot(jnp.dot(A, C0, preferred_element_type=f32), W1,
                     preferred_element_type=f32)
        b = jnp.dot(A, dinv_col, preferred_element_type=f32) + dinv_col

        U1 = jnp.dot(u0_acc[...], W1, preferred_element_type=f32)

        h_blocks = []
        va_blocks = []
        vcol = v_row.reshape(DIM_GCN, 1)
        for i in range(N_EXP):
            bi = jax.lax.slice(b, (i, 0), (i + 1, 1))
            c1i = jax.lax.slice(C1, (i, 0), (i + 1, DIM_GCN))
            h_blocks.append(U1 * bi + c1i)
            arow = jax.lax.slice(A, (0, i), (N_EXP, i + 1))
            va_blocks.append(vcol * arow.reshape(1, N_EXP))
        h_blocks.append(U1)
        va_blocks.append(vcol * dinv_row)
        Hcat = jnp.maximum(jnp.concatenate(h_blocks, axis=1), 0.0)
        VA = jnp.concatenate(va_blocks, axis=0)

        S = jnp.dot(Hcat, VA, preferred_element_type=f32)

        m = jnp.max(S, axis=1, keepdims=True)
        e = jnp.exp(S - m)
        out_ref[...] = e / jnp.sum(e, axis=1, keepdims=True)


def kernel(x, X, W_mlp, W0, W1, W2, W_proj):
    ori_shape = x.shape[:-1]
    x2 = x.reshape(-1, DIM)
    XT = X.T
    out = pl.pallas_call(
        _moegnn_body,
        grid=(NQ,),
        in_specs=[
            pl.BlockSpec((NTOK, DIM), lambda j: (0, 0)),
            pl.BlockSpec((N_EXP, DIM), lambda j: (0, 0)),
            pl.BlockSpec((QROWS, DIM), lambda j: (j, 0)),
            pl.BlockSpec((DIM, DIM_GCN), lambda j: (0, 0)),
            pl.BlockSpec((DIM_GCN, DIM_GCN), lambda j: (0, 0)),
            pl.BlockSpec((DIM_GCN, DIM), lambda j: (0, 0)),
            pl.BlockSpec((DIM, 1), lambda j: (0, 0)),
        ],
        out_specs=pl.BlockSpec((NTOK, N_EXP), lambda j: (0, 0)),
        out_shape=jax.ShapeDtypeStruct((NTOK, N_EXP), jnp.float32),
        scratch_shapes=[
            pltpu.VMEM((NTOK, DIM_GCN), jnp.float32),
            pltpu.VMEM((N_EXP, DIM), jnp.float32),
            pltpu.VMEM((N_EXP, DIM_GCN), jnp.float32),
        ],
    )(x2, XT, W_mlp, W0, W1, W2, W_proj)
    return out.reshape(*ori_shape, N_EXP)


# speedup vs baseline: 1.1261x; 1.1261x over previous
import jax
import jax.numpy as jnp
from jax.experimental import pallas as pl
from jax.experimental.pallas import tpu as pltpu

DIM = 1024
N_EXP = 16
DIM_GCN = 256
THRESH = 0.8
NTOK = 256
NQ = 2
QROWS = DIM // NQ
_HI = jax.lax.Precision.HIGHEST


def _moegnn_body(x_ref, XT_ref, Wm_ref, W0_ref, W1_ref, W2_ref, Wp_ref,
                 out_ref, u0_acc, e_s, ew0_acc):
    f32 = jnp.float32
    j = pl.program_id(0)

    Wm_q = Wm_ref[...]
    W0_q = W0_ref[pl.ds(j * QROWS, QROWS), :]
    xf_q = jnp.maximum(
        jax.lax.dot_general(x_ref[...], Wm_q, (((1,), (1,)), ((), ())),
                            preferred_element_type=f32), 0.0)
    p_u0 = jnp.dot(xf_q, W0_q, preferred_element_type=f32)
    e_q = jnp.maximum(
        jax.lax.dot_general(XT_ref[...], Wm_q, (((1,), (1,)), ((), ())),
                            preferred_element_type=f32), 0.0)
    e_s[:, pl.ds(j * QROWS, QROWS)] = e_q
    p_ew0 = jnp.dot(e_q, W0_q, preferred_element_type=f32)

    @pl.when(j == 0)
    def _init():
        u0_acc[...] = p_u0
        ew0_acc[...] = p_ew0

    @pl.when(j > 0)
    def _accum():
        u0_acc[...] += p_u0
        ew0_acc[...] += p_ew0

    @pl.when(j == NQ - 1)
    def _tail():
        W1 = W1_ref[...]
        ee = e_s[...]

        v_row = jax.lax.dot_general(Wp_ref[...], W2_ref[...],
                                    (((0,), (1,)), ((), ())),
                                    preferred_element_type=f32)

        G = jax.lax.dot_general(ee, ee, (((1,), (1,)), ((), ())),
                                preferred_element_type=f32)
        nrm2 = jnp.sum(ee * ee, axis=1, keepdims=True)
        nrm_col = jnp.maximum(jnp.sqrt(nrm2), 1e-8)
        nrm_row = jnp.transpose(nrm_col)

        ri = jax.lax.broadcasted_iota(jnp.int32, (N_EXP, N_EXP), 0)
        ci = jax.lax.broadcasted_iota(jnp.int32, (N_EXP, N_EXP), 1)
        eye = jnp.where(ri == ci, 1.0, 0.0)
        cos = G / (nrm_col * nrm_row)
        ind = (cos > THRESH).astype(f32)
        lower = jnp.where(ri > ci, ind, 0.0)
        upper = jnp.where(ri < ci, ind, 0.0)
        dinv_col = jax.lax.rsqrt(2.0 + jnp.sum(lower, axis=1, keepdims=True))
        dinv_row = jax.lax.rsqrt(2.0 + jnp.sum(upper, axis=0, keepdims=True))
        A = dinv_col * dinv_row * (lower + eye)

        C0 = jnp.dot(A, ew0_acc[...], preferred_element_type=f32)
        C1 = jnp.dot(jnp.dot(A, C0, preferred_element_type=f32), W1,
                     preferred_element_type=f32)
        b = jnp.dot(A, dinv_col, preferred_element_type=f32) + dinv_col

        U1 = jnp.dot(u0_acc[...], W1, preferred_element_type=f32)

        h_blocks = []
        va_blocks = []
        vcol = v_row.reshape(DIM_GCN, 1)
        for i in range(N_EXP):
            bi = jax.lax.slice(b, (i, 0), (i + 1, 1))
            c1i = jax.lax.slice(C1, (i, 0), (i + 1, DIM_GCN))
            h_blocks.append(U1 * bi + c1i)
            arow = jax.lax.slice(A, (0, i), (N_EXP, i + 1))
            va_blocks.append(vcol * arow.reshape(1, N_EXP))
        h_blocks.append(U1)
        va_blocks.append(vcol * dinv_row)
        Hcat = jnp.maximum(jnp.concatenate(h_blocks, axis=1), 0.0)
        VA = jnp.concatenate(va_blocks, axis=0)

        S = jnp.dot(Hcat, VA, preferred_element_type=f32)

        m = jnp.max(S, axis=1, keepdims=True)
        e = jnp.exp(S - m)
        out_ref[...] = e / jnp.sum(e, axis=1, keepdims=True)


def kernel(x, X, W_mlp, W0, W1, W2, W_proj):
    ori_shape = x.shape[:-1]
    x2 = x.reshape(-1, DIM)
    XT = X.T
    out = pl.pallas_call(
        _moegnn_body,
        grid=(NQ,),
        in_specs=[
            pl.BlockSpec((NTOK, DIM), lambda j: (0, 0)),
            pl.BlockSpec((N_EXP, DIM), lambda j: (0, 0)),
            pl.BlockSpec((QROWS, DIM), lambda j: (j, 0)),
            pl.BlockSpec((DIM, DIM_GCN), lambda j: (0, 0)),
            pl.BlockSpec((DIM_GCN, DIM_GCN), lambda j: (0, 0)),
            pl.BlockSpec((DIM_GCN, DIM), lambda j: (0, 0)),
            pl.BlockSpec((DIM, 1), lambda j: (0, 0)),
        ],
        out_specs=pl.BlockSpec((NTOK, N_EXP), lambda j: (0, 0)),
        out_shape=jax.ShapeDtypeStruct((NTOK, N_EXP), jnp.float32),
        scratch_shapes=[
            pltpu.VMEM((NTOK, DIM_GCN), jnp.float32),
            pltpu.VMEM((N_EXP, DIM), jnp.float32),
            pltpu.VMEM((N_EXP, DIM_GCN), jnp.float32),
        ],
    )(x2, XT, W_mlp, W0, W1, W2, W_proj)
    return out.reshape(*ori_shape, N_EXP)
